# Initial kernel scaffold; baseline (speedup 1.0000x reference)
import jax
import jax.numpy as jnp
from jax.experimental import pallas as pl


def kernel(feature, input, adj_indices, adj_values, alpha, weight):
    raise NotImplementedError("write your pallas kernel here")


# speedup vs baseline: 4.5324x; 4.5324x over previous
import functools

import jax
import jax.numpy as jnp
from jax import lax
from jax.experimental import pallas as pl
from jax.experimental.pallas import tpu as pltpu
from jax.experimental.pallas import tpu_sc as plsc

N = 10000
D = 128
E = 320000

NC = 2
NS = 16
NW = NC * NS
E_PER_W = E // NW
C = 80
CHUNKS = E_PER_W // C
N_PAD = 10240
ROWS_PER_TILE = N_PAD // NS
ZROWS = 128


def _make_sc_spmm():
    mesh = plsc.VectorSubcoreMesh(core_axis_name="c", subcore_axis_name="s")

    @functools.partial(
        pl.kernel,
        mesh=mesh,
        out_type=jax.ShapeDtypeStruct((NC, N_PAD, D), jnp.float32),
        scratch_types=[
            pltpu.VMEM_SHARED((N_PAD, D), jnp.float32),
            pltpu.VMEM((ZROWS, D), jnp.float32),
            pltpu.VMEM((C,), jnp.int32),
            pltpu.VMEM((C,), jnp.int32),
            pltpu.VMEM((C,), jnp.float32),
            pltpu.VMEM((C, D), jnp.float32),
            pltpu.SemaphoreType.DMA,
        ],
    )
    def sc_spmm(input_hbm, col_hbm, row_hbm, val_hbm, out_hbm,
                acc, zbuf, colv, rowv, valv, rows, sem):
        cid = lax.axis_index("c")
        sid = lax.axis_index("s")

        def zero_row(i, carry):
            for d8 in range(D // 16):
                zbuf[i, pl.ds(d8 * 16, 16)] = jnp.zeros((16,), jnp.float32)
            return carry

        lax.fori_loop(0, ZROWS, zero_row, 0)
        for k in range(ROWS_PER_TILE // ZROWS):
            pltpu.sync_copy(
                zbuf, acc.at[pl.ds(sid * ROWS_PER_TILE + k * ZROWS, ZROWS)])
        plsc.subcore_barrier()

        base_w = (sid * NC + cid) * E_PER_W

        def chunk_body(ci, carry):
            base = base_w + ci * C
            pltpu.sync_copy(col_hbm.at[pl.ds(base, C)], colv)
            pltpu.sync_copy(row_hbm.at[pl.ds(base, C)], rowv)
            pltpu.sync_copy(val_hbm.at[pl.ds(base, C)], valv)
            pltpu.async_copy(input_hbm.at[colv], rows, sem).wait()

            def group_body(g, inner):
                vv = valv[pl.ds(g * 16, 16)]
                for j in range(16):
                    v = vv[j]
                    e = g * 16 + j
                    for d8 in range(D // 16):
                        sl = pl.ds(d8 * 16, 16)
                        rows[e, sl] = rows[e, sl] * v
                return inner

            lax.fori_loop(0, C // 16, group_body, 0)
            pltpu.sync_copy(rows, acc.at[rowv], add=True)
            return carry

        lax.fori_loop(0, CHUNKS, chunk_body, 0)
        plsc.subcore_barrier()

        pltpu.sync_copy(
            acc.at[pl.ds(sid * ROWS_PER_TILE, ROWS_PER_TILE)],
            out_hbm.at[cid, pl.ds(sid * ROWS_PER_TILE, ROWS_PER_TILE)])

    return sc_spmm


_sc_spmm = _make_sc_spmm()

_BLK = 1000


def _blend_body(alpha_ref, f_ref, p0_ref, p1_ref, o_ref):
    a = alpha_ref[0]
    o_ref[...] = (1.0 - a) * (p0_ref[0] + p1_ref[0]) + a * f_ref[...]


def _blend(alpha, feature, partial):
    return pl.pallas_call(
        _blend_body,
        grid=(N // _BLK,),
        in_specs=[
            pl.BlockSpec(memory_space=pltpu.SMEM),
            pl.BlockSpec((_BLK, D), lambda i: (i, 0)),
            pl.BlockSpec((1, _BLK, D), lambda i: (0, i, 0)),
            pl.BlockSpec((1, _BLK, D), lambda i: (1, i, 0)),
        ],
        out_specs=pl.BlockSpec((_BLK, D), lambda i: (i, 0)),
        out_shape=jax.ShapeDtypeStruct((N, D), jnp.float32),
    )(alpha, feature, partial, partial)


def kernel(feature, input, adj_indices, adj_values, alpha, weight):
    del weight
    row = adj_indices[0]
    col = adj_indices[1]
    partial = _sc_spmm(input, col, row, adj_values)
    return _blend(jnp.reshape(alpha, (1,)), feature, partial)


# speedup vs baseline: 11.2238x; 2.4764x over previous
import functools

import jax
import jax.numpy as jnp
from jax import lax
from jax.experimental import pallas as pl
from jax.experimental.pallas import tpu as pltpu
from jax.experimental.pallas import tpu_sc as plsc

N = 10000
D = 128
E = 320000

NC = 2
NS = 16
NW = NC * NS
E_PER_W = E // NW
C = 80
CHUNKS = E_PER_W // C
N_PAD = 10240
ROWS_PER_TILE = N_PAD // NS


def _make_sc_spmm():
    mesh = plsc.VectorSubcoreMesh(core_axis_name="c", subcore_axis_name="s")

    @functools.partial(
        pl.kernel,
        mesh=mesh,
        out_type=jax.ShapeDtypeStruct((NC, N_PAD, D), jnp.float32),
        scratch_types=[
            pltpu.VMEM_SHARED((N_PAD, D), jnp.float32),
            pltpu.VMEM((CHUNKS, C), jnp.int32),
            pltpu.VMEM((C,), jnp.int32),
            pltpu.VMEM((C,), jnp.int32),
            pltpu.VMEM((C,), jnp.float32),
            pltpu.VMEM((C,), jnp.float32),
            pltpu.VMEM((C, D), jnp.float32),
            pltpu.VMEM((C, D), jnp.float32),
            pltpu.SemaphoreType.DMA,
            pltpu.SemaphoreType.DMA,
            pltpu.SemaphoreType.DMA,
            pltpu.SemaphoreType.DMA,
            pltpu.SemaphoreType.DMA,
            pltpu.SemaphoreType.DMA,
        ],
    )
    def sc_spmm(input_hbm, col_hbm, row_hbm, val_hbm, out_hbm,
                acc, colv, rowb0, rowb1, valb0, valb1, rows0, rows1,
                sem0, sem1, semv0, semv1, semr0, semr1):
        cid = lax.axis_index("c")
        sid = lax.axis_index("s")
        w = sid * NC + cid
        rows_bufs = (rows0, rows1)
        row_bufs = (rowb0, rowb1)
        val_bufs = (valb0, valb1)
        sems = (sem0, sem1)
        semsv = (semv0, semv1)
        semsr = (semr0, semr1)

        def zero_row(i, carry):
            for d8 in range(D // 16):
                rows0[i, pl.ds(d8 * 16, 16)] = jnp.zeros((16,), jnp.float32)
            return carry

        lax.fori_loop(0, C, zero_row, 0)
        for k in range(ROWS_PER_TILE // C):
            pltpu.sync_copy(rows0, acc.at[pl.ds(sid * ROWS_PER_TILE + k * C, C)])
        plsc.subcore_barrier()

        pltpu.sync_copy(col_hbm.at[w], colv)

        def start_gather(ci, b):
            pltpu.make_async_copy(
                val_hbm.at[pl.ds(w * E_PER_W + ci * C, C)],
                val_bufs[b], semsv[b]).start()
            pltpu.make_async_copy(
                row_hbm.at[pl.ds(w * E_PER_W + ci * C, C)],
                row_bufs[b], semsr[b]).start()
            pltpu.make_async_copy(
                input_hbm.at[colv.at[ci]], rows_bufs[b], sems[b]).start()

        def wait_gather(ci, b):
            pltpu.make_async_copy(
                val_hbm.at[pl.ds(w * E_PER_W + ci * C, C)],
                val_bufs[b], semsv[b]).wait()
            pltpu.make_async_copy(
                row_hbm.at[pl.ds(w * E_PER_W + ci * C, C)],
                row_bufs[b], semsr[b]).wait()
            pltpu.make_async_copy(
                input_hbm.at[colv.at[ci]], rows_bufs[b], sems[b]).wait()

        def process(ci, b):
            rows = rows_bufs[b]

            def group_body(g, inner):
                vv = val_bufs[b][pl.ds(g * 16, 16)]
                for j in range(16):
                    v = vv[j]
                    e = g * 16 + j
                    for d8 in range(D // 16):
                        sl = pl.ds(d8 * 16, 16)
                        rows[e, sl] = rows[e, sl] * v
                return inner

            lax.fori_loop(0, C // 16, group_body, 0)
            pltpu.sync_copy(rows, acc.at[row_bufs[b]], add=True)

        start_gather(0, 0)

        def pipe_body(cj, carry):
            for b in range(2):
                ci = 2 * cj + b
                start_gather(ci + 1, 1 - b)
                wait_gather(ci, b)
                process(ci, b)
            return carry

        lax.fori_loop(0, (CHUNKS - 1) // 2, pipe_body, 0)
        wait_gather(CHUNKS - 1, 0)
        process(CHUNKS - 1, 0)
        plsc.subcore_barrier()

        pltpu.sync_copy(
            acc.at[pl.ds(sid * ROWS_PER_TILE, ROWS_PER_TILE)],
            out_hbm.at[cid, pl.ds(sid * ROWS_PER_TILE, ROWS_PER_TILE)])

    return sc_spmm


_sc_spmm = _make_sc_spmm()

_BLK = 1000


def _blend_body(alpha_ref, f_ref, p0_ref, p1_ref, o_ref):
    a = alpha_ref[0]
    o_ref[...] = (1.0 - a) * (p0_ref[0] + p1_ref[0]) + a * f_ref[...]


def _blend(alpha, feature, partial):
    return pl.pallas_call(
        _blend_body,
        grid=(N // _BLK,),
        in_specs=[
            pl.BlockSpec(memory_space=pltpu.SMEM),
            pl.BlockSpec((_BLK, D), lambda i: (i, 0)),
            pl.BlockSpec((1, _BLK, D), lambda i: (0, i, 0)),
            pl.BlockSpec((1, _BLK, D), lambda i: (1, i, 0)),
        ],
        out_specs=pl.BlockSpec((_BLK, D), lambda i: (i, 0)),
        out_shape=jax.ShapeDtypeStruct((N, D), jnp.float32),
    )(alpha, feature, partial, partial)


def kernel(feature, input, adj_indices, adj_values, alpha, weight):
    del weight
    row = adj_indices[0]
    col = jnp.reshape(adj_indices[1], (NW, CHUNKS, C))
    partial = _sc_spmm(input, col, row, adj_values)
    return _blend(jnp.reshape(alpha, (1,)), feature, partial)
